# baseline (speedup 1.0000x reference)
import functools

import jax
import jax.numpy as jnp
from jax import lax
from jax.experimental import pallas as pl
from jax.experimental.pallas import tpu as pltpu
from jax.experimental.pallas import tpu_sc as plsc

K = 27
C = 128
NC = 2
NS = 16
NW = NC * NS
CHUNK = 128
FB = 4

_mesh = lambda: plsc.VectorSubcoreMesh(core_axis_name="c", subcore_axis_name="s")


def _sc_gather(table, idx2, e_pad):
    nch = e_pad // CHUNK
    cpw0 = nch // NS
    cpw1 = nch // NS - cpw0

    def body(tab_ref, idx_ref, out_ref, idxv, rows, sg):
        c = lax.axis_index("c")
        s = lax.axis_index("s")

        def flow(base, cpw):
            pltpu.sync_copy(idx_ref.at[pl.ds(base, cpw)], idxv.at[pl.ds(0, cpw)])

            def step(i, carry):
                cps = [
                    pltpu.async_copy(tab_ref.at[idxv.at[i * FB + b]], rows.at[b], sg)
                    for b in range(FB)
                ]
                for b in range(FB):
                    cps[b].wait()
                for b in range(FB):
                    pltpu.sync_copy(
                        rows.at[b],
                        out_ref.at[pl.ds((base + i * FB + b) * CHUNK, CHUNK)],
                    )
                return carry

            lax.fori_loop(0, cpw // FB, step, 0)

        @pl.when(c == 1)
        def _():
            flow(s * cpw0, cpw0)

        if cpw1 > 0:

            @pl.when(c == 0)
            def _():
                flow(NS * cpw0 + s * cpw1, cpw1)

    return pl.kernel(
        body,
        out_type=jax.ShapeDtypeStruct((e_pad, C), jnp.float32),
        mesh=_mesh(),
        scratch_types=[
            pltpu.VMEM((cpw0, CHUNK), jnp.int32),
            pltpu.VMEM((FB, CHUNK, C), jnp.float32),
            pltpu.SemaphoreType.DMA,
        ],
    )(table, idx2)


def _sc_scatter(msg, dst2, e_pad, n_out):
    nchunks = e_pad // CHUNK
    cpt = nchunks // NS
    rb = NS * 512
    nb = n_out // rb
    rpt = rb // NS

    def body(msg_ref, dst_ref, out_ref, zv, mv0, mv1, iv2, dstv,
             bmn, bmx, hl, spm, sg0, sg1):
        c = lax.axis_index("c")
        s = lax.axis_index("s")
        pltpu.sync_copy(dst_ref.at[pl.ds(s * cpt, cpt)], dstv)

        def z1(i, carry):
            for q in range(8):
                zv[i, pl.ds(q * 16, 16)] = jnp.zeros((16,), jnp.float32)
            return carry

        lax.fori_loop(0, CHUNK, z1, 0)

        def mm(j, carry):
            mn = jnp.int32(2147483647)
            mx = jnp.int32(-1)
            for q in range(8):
                lv = dstv[j, pl.ds(q * 16, 16)]
                mn = jnp.minimum(mn, jnp.min(lv))
                mx = jnp.maximum(mx, jnp.max(lv))
            bmn[j] = mn
            bmx[j] = mx
            return carry

        lax.fori_loop(0, cpt, mm, 0)

        for bi in range(-(-nb // NC)):
            b = bi * NC + c
            base = b * rb
            live = b < nb
            for z in range(rpt // CHUNK):
                pltpu.sync_copy(zv, spm.at[pl.ds(s * rpt + z * CHUNK, CHUNK)])
            plsc.subcore_barrier()

            def hlist(j, m):
                hit = jnp.logical_and(bmn[j] < base + rb, bmx[j] >= base)
                hl[m] = j
                return m + jnp.where(hit, jnp.int32(1), jnp.int32(0))

            m = lax.fori_loop(0, cpt, hlist, jnp.int32(0))

            def fire(i, mv, sg):
                ch = s * cpt + hl[i]
                pltpu.async_copy(msg_ref.at[pl.ds(ch * CHUNK, CHUNK)], mv, sg)

            @pl.when(m > 0)
            def _():
                fire(0, mv0, sg0)

            def proc(i, mv, sg, nmv, nsg):
                @pl.when(i + 1 < m)
                def _():
                    fire(jnp.minimum(i + 1, cpt - 1), nmv, nsg)

                ch = s * cpt + hl[i]
                pltpu.make_async_copy(
                    msg_ref.at[pl.ds(ch * CHUNK, CHUNK)], mv, sg
                ).wait()
                j = hl[i]
                for q in range(8):
                    lv = dstv[j, pl.ds(q * 16, 16)] - base
                    ok = jnp.logical_and(lv >= 0, lv < rb)
                    iv2[pl.ds(q * 16, 16)] = jnp.where(ok, lv, rb)
                pltpu.sync_copy(mv, spm.at[iv2], add=True)

            def pbody(i, carry):
                @pl.when(i % 2 == 0)
                def _():
                    proc(i, mv0, sg0, mv1, sg1)

                @pl.when(i % 2 == 1)
                def _():
                    proc(i, mv1, sg1, mv0, sg0)

                return carry

            lax.fori_loop(0, m, pbody, 0)
            plsc.subcore_barrier()

            def wb(w, carry):
                r0 = s * rpt + w * CHUNK
                pltpu.sync_copy(
                    spm.at[pl.ds(r0, CHUNK)],
                    out_ref.at[pl.ds(base + r0, CHUNK)],
                )
                return carry

            @pl.when(live)
            def _():
                lax.fori_loop(0, rpt // CHUNK, wb, 0)

            plsc.subcore_barrier()

    return pl.kernel(
        body,
        out_type=jax.ShapeDtypeStruct((n_out, C), jnp.float32),
        mesh=_mesh(),
        compiler_params=pltpu.CompilerParams(needs_layout_passes=False),
        scratch_types=[
            pltpu.VMEM((CHUNK, C), jnp.float32),
            pltpu.VMEM((CHUNK, C), jnp.float32),
            pltpu.VMEM((CHUNK, C), jnp.float32),
            pltpu.VMEM((CHUNK,), jnp.int32),
            pltpu.VMEM((cpt, CHUNK), jnp.int32),
            pltpu.SMEM((cpt,), jnp.int32),
            pltpu.SMEM((cpt,), jnp.int32),
            pltpu.SMEM((cpt + 1,), jnp.int32),
            pltpu.VMEM_SHARED((rb + 8, C), jnp.float32),
            pltpu.SemaphoreType.DMA,
            pltpu.SemaphoreType.DMA,
        ],
    )(msg, dst2)


def _mm_body(ends_ref, *refs, norm, tb):
    if norm:
        x_ref, w_ref, mi_ref, o_ref = refs
    else:
        x_ref, w_ref, o_ref = refs
    r0 = pl.program_id(0) * tb
    x = x_ref[...]
    if norm:
        t = (x - mi_ref[0:1, :]) * mi_ref[1:2, :]
        x = jnp.where(t >= 0, t, 0.2 * t)

    def cnt(j, a):
        return a + jnp.where(ends_ref[j] <= r0, 1, 0)

    k0 = lax.fori_loop(0, K, cnt, 0)
    rows = r0 + lax.broadcasted_iota(jnp.int32, (tb, 1), 0)

    def start_of(k):
        return jnp.where(k == 0, 0, ends_ref[jnp.maximum(k, 1) - 1])

    def wcond(st):
        k, _ = st
        return jnp.logical_and(k < K, start_of(jnp.minimum(k, K - 1)) < r0 + tb)

    def wbody(st):
        k, acc = st
        kc = jnp.minimum(k, K - 1)
        m = (rows >= start_of(kc)) & (rows < ends_ref[kc])
        xm = jnp.where(m, x, 0.0).astype(jnp.bfloat16)
        acc = acc + jnp.dot(xm, w_ref[kc], preferred_element_type=jnp.float32)
        return (k + 1, acc)

    _, acc = lax.while_loop(wcond, wbody, (k0, jnp.zeros((tb, C), jnp.float32)))
    o_ref[...] = acc


def _tc_ragged_mm(xg, w, ends, mi, e_pad):
    tb = 1024
    norm = mi is not None
    w = w.astype(jnp.bfloat16)
    in_specs = [
        pl.BlockSpec((tb, C), lambda i, e: (i, 0)),
        pl.BlockSpec((K, C, C), lambda i, e: (0, 0, 0)),
    ]
    operands = [xg, w]
    if norm:
        in_specs.append(pl.BlockSpec((8, C), lambda i, e: (0, 0)))
        operands.append(mi)
    grid_spec = pltpu.PrefetchScalarGridSpec(
        num_scalar_prefetch=1,
        grid=(e_pad // tb,),
        in_specs=in_specs,
        out_specs=pl.BlockSpec((tb, C), lambda i, e: (i, 0)),
    )
    return pl.pallas_call(
        functools.partial(_mm_body, norm=norm, tb=tb),
        grid_spec=grid_spec,
        out_shape=jax.ShapeDtypeStruct((e_pad, C), jnp.float32),
    )(ends, *operands)


def _mom_body(h_ref, o_ref, acc_ref, *, nsteps):
    i = pl.program_id(0)
    h = h_ref[...]
    blk = jnp.concatenate(
        [
            jnp.sum(h, axis=0, keepdims=True),
            jnp.sum(h * h, axis=0, keepdims=True),
            jnp.zeros((6, C), jnp.float32),
        ],
        axis=0,
    )

    @pl.when(i == 0)
    def _():
        acc_ref[...] = blk

    @pl.when(i > 0)
    def _():
        acc_ref[...] = acc_ref[...] + blk

    @pl.when(i == nsteps - 1)
    def _():
        o_ref[...] = acc_ref[...]


def _tc_moments(h, n_out):
    tb = 2048
    nsteps = n_out // tb
    return pl.pallas_call(
        functools.partial(_mom_body, nsteps=nsteps),
        grid=(nsteps,),
        in_specs=[pl.BlockSpec((tb, C), lambda i: (i, 0))],
        out_specs=pl.BlockSpec((8, C), lambda i: (0, 0)),
        out_shape=jax.ShapeDtypeStruct((8, C), jnp.float32),
        scratch_shapes=[pltpu.VMEM((8, C), jnp.float32)],
    )(h)


def _lr_body(x_ref, o_ref):
    x = x_ref[...]
    o_ref[...] = jnp.where(x >= 0, x, 0.2 * x)


def _tc_leaky(o, n):
    tb = 1000
    return pl.pallas_call(
        _lr_body,
        grid=(n // tb,),
        in_specs=[pl.BlockSpec((tb, C), lambda i: (i, 0))],
        out_specs=pl.BlockSpec((tb, C), lambda i: (i, 0)),
        out_shape=jax.ShapeDtypeStruct((n, C), jnp.float32),
    )(o)


def kernel(x_feat, W1, W2, src, dst, counts):
    n = x_feat.shape[0]
    e = src.shape[0]
    e_pad = -(-e // (NW * CHUNK * 8)) * (NW * CHUNK * 8)
    n_out = -(-(n + 1) // (NS * 512)) * (NS * 512)

    src = src.astype(jnp.int32)
    dst = dst.astype(jnp.int32)
    ends = jnp.cumsum(counts).astype(jnp.int32)
    srcp = jnp.concatenate([src, jnp.zeros((e_pad - e,), jnp.int32)])
    srcp = srcp.reshape(e_pad // CHUNK, CHUNK)
    dstp = jnp.concatenate([dst, jnp.full((e_pad - e,), n, jnp.int32)])
    dstp = dstp.reshape(e_pad // CHUNK, CHUNK)

    xg1 = _sc_gather(x_feat, srcp, e_pad)
    msg1 = _tc_ragged_mm(xg1, W1, ends, None, e_pad)
    h = _sc_scatter(msg1, dstp, e_pad, n_out)
    mom = _tc_moments(h, n_out)
    mean = mom[0] / n
    inv = lax.rsqrt(mom[1] / n - mean * mean + 1e-5)
    mi = jnp.zeros((8, C), jnp.float32).at[0].set(mean).at[1].set(inv)
    xg2 = _sc_gather(h, srcp, e_pad)
    msg2 = _tc_ragged_mm(xg2, W2, ends, mi, e_pad)
    o = _sc_scatter(msg2, dstp, e_pad, n_out)
    return _tc_leaky(o, n)


# speedup vs baseline: 1.0881x; 1.0881x over previous
import functools

import jax
import jax.numpy as jnp
from jax import lax
from jax.experimental import pallas as pl
from jax.experimental.pallas import tpu as pltpu
from jax.experimental.pallas import tpu_sc as plsc

K = 27
C = 128
NC = 2
NS = 16
NW = NC * NS
CHUNK = 128
FB = 4

_mesh = lambda: plsc.VectorSubcoreMesh(core_axis_name="c", subcore_axis_name="s")


def _sc_gather(table, idx2, e_pad):
    nch = e_pad // CHUNK
    cpw0 = nch * 6 // (7 * NS) // 8 * 8
    cpw1 = nch // NS - cpw0

    def body(tab_ref, idx_ref, out_ref, idxv, rows, sg):
        c = lax.axis_index("c")
        s = lax.axis_index("s")

        def flow(base, cpw):
            pltpu.sync_copy(idx_ref.at[pl.ds(base, cpw)], idxv.at[pl.ds(0, cpw)])

            def step(i, carry):
                cps = [
                    pltpu.async_copy(tab_ref.at[idxv.at[i * FB + b]], rows.at[b], sg)
                    for b in range(FB)
                ]
                for b in range(FB):
                    cps[b].wait()
                for b in range(FB):
                    pltpu.sync_copy(
                        rows.at[b],
                        out_ref.at[pl.ds((base + i * FB + b) * CHUNK, CHUNK)],
                    )
                return carry

            lax.fori_loop(0, cpw // FB, step, 0)

        @pl.when(c == 1)
        def _():
            flow(s * cpw0, cpw0)

        if cpw1 > 0:

            @pl.when(c == 0)
            def _():
                flow(NS * cpw0 + s * cpw1, cpw1)

    return pl.kernel(
        body,
        out_type=jax.ShapeDtypeStruct((e_pad, C), jnp.float32),
        mesh=_mesh(),
        scratch_types=[
            pltpu.VMEM((cpw0, CHUNK), jnp.int32),
            pltpu.VMEM((FB, CHUNK, C), jnp.float32),
            pltpu.SemaphoreType.DMA,
        ],
    )(table, idx2)


def _sc_scatter(msg, dst2, e_pad, n_out):
    nchunks = e_pad // CHUNK
    cpt = nchunks // NS
    rb = NS * 512
    nb = n_out // rb
    rpt = rb // NS

    def body(msg_ref, dst_ref, out_ref, zv, mv0, mv1, iv2, dstv,
             bmn, bmx, hl, spm, sg0, sg1):
        c = lax.axis_index("c")
        s = lax.axis_index("s")
        pltpu.sync_copy(dst_ref.at[pl.ds(s * cpt, cpt)], dstv)

        def z1(i, carry):
            for q in range(8):
                zv[i, pl.ds(q * 16, 16)] = jnp.zeros((16,), jnp.float32)
            return carry

        lax.fori_loop(0, CHUNK, z1, 0)

        def mm(j, carry):
            mn = jnp.int32(2147483647)
            mx = jnp.int32(-1)
            for q in range(8):
                lv = dstv[j, pl.ds(q * 16, 16)]
                mn = jnp.minimum(mn, jnp.min(lv))
                mx = jnp.maximum(mx, jnp.max(lv))
            bmn[j] = mn
            bmx[j] = mx
            return carry

        lax.fori_loop(0, cpt, mm, 0)

        for bi in range(-(-nb // NC)):
            b = bi * NC + c
            base = b * rb
            live = b < nb
            for z in range(rpt // CHUNK):
                pltpu.sync_copy(zv, spm.at[pl.ds(s * rpt + z * CHUNK, CHUNK)])
            plsc.subcore_barrier()

            def hlist(j, m):
                hit = jnp.logical_and(bmn[j] < base + rb, bmx[j] >= base)
                hl[m] = j
                return m + jnp.where(hit, jnp.int32(1), jnp.int32(0))

            m = lax.fori_loop(0, cpt, hlist, jnp.int32(0))

            def fire(i, mv, sg):
                ch = s * cpt + hl[i]
                pltpu.async_copy(msg_ref.at[pl.ds(ch * CHUNK, CHUNK)], mv, sg)

            @pl.when(m > 0)
            def _():
                fire(0, mv0, sg0)

            def proc(i, mv, sg, nmv, nsg):
                @pl.when(i + 1 < m)
                def _():
                    fire(jnp.minimum(i + 1, cpt - 1), nmv, nsg)

                ch = s * cpt + hl[i]
                pltpu.make_async_copy(
                    msg_ref.at[pl.ds(ch * CHUNK, CHUNK)], mv, sg
                ).wait()
                j = hl[i]
                for q in range(8):
                    lv = dstv[j, pl.ds(q * 16, 16)] - base
                    ok = jnp.logical_and(lv >= 0, lv < rb)
                    iv2[pl.ds(q * 16, 16)] = jnp.where(ok, lv, rb)
                pltpu.sync_copy(mv, spm.at[iv2], add=True)

            def pbody(i, carry):
                @pl.when(i % 2 == 0)
                def _():
                    proc(i, mv0, sg0, mv1, sg1)

                @pl.when(i % 2 == 1)
                def _():
                    proc(i, mv1, sg1, mv0, sg0)

                return carry

            lax.fori_loop(0, m, pbody, 0)
            plsc.subcore_barrier()

            def wb(w, carry):
                r0 = s * rpt + w * CHUNK
                pltpu.sync_copy(
                    spm.at[pl.ds(r0, CHUNK)],
                    out_ref.at[pl.ds(base + r0, CHUNK)],
                )
                return carry

            @pl.when(live)
            def _():
                lax.fori_loop(0, rpt // CHUNK, wb, 0)

            plsc.subcore_barrier()

    return pl.kernel(
        body,
        out_type=jax.ShapeDtypeStruct((n_out, C), jnp.float32),
        mesh=_mesh(),
        compiler_params=pltpu.CompilerParams(needs_layout_passes=False),
        scratch_types=[
            pltpu.VMEM((CHUNK, C), jnp.float32),
            pltpu.VMEM((CHUNK, C), jnp.float32),
            pltpu.VMEM((CHUNK, C), jnp.float32),
            pltpu.VMEM((CHUNK,), jnp.int32),
            pltpu.VMEM((cpt, CHUNK), jnp.int32),
            pltpu.SMEM((cpt,), jnp.int32),
            pltpu.SMEM((cpt,), jnp.int32),
            pltpu.SMEM((cpt + 1,), jnp.int32),
            pltpu.VMEM_SHARED((rb + 8, C), jnp.float32),
            pltpu.SemaphoreType.DMA,
            pltpu.SemaphoreType.DMA,
        ],
    )(msg, dst2)


def _mm_body(ends_ref, *refs, norm, tb):
    if norm:
        x_ref, w_ref, mi_ref, o_ref = refs
    else:
        x_ref, w_ref, o_ref = refs
    r0 = pl.program_id(0) * tb
    x = x_ref[...]
    if norm:
        t = (x - mi_ref[0:1, :]) * mi_ref[1:2, :]
        x = jnp.where(t >= 0, t, 0.2 * t)

    def cnt(j, a):
        return a + jnp.where(ends_ref[j] <= r0, 1, 0)

    k0 = lax.fori_loop(0, K, cnt, 0)
    rows = r0 + lax.broadcasted_iota(jnp.int32, (tb, 1), 0)

    def start_of(k):
        return jnp.where(k == 0, 0, ends_ref[jnp.maximum(k, 1) - 1])

    def wcond(st):
        k, _ = st
        return jnp.logical_and(k < K, start_of(jnp.minimum(k, K - 1)) < r0 + tb)

    def wbody(st):
        k, acc = st
        kc = jnp.minimum(k, K - 1)
        m = (rows >= start_of(kc)) & (rows < ends_ref[kc])
        xm = jnp.where(m, x, 0.0).astype(jnp.bfloat16)
        acc = acc + jnp.dot(xm, w_ref[kc], preferred_element_type=jnp.float32)
        return (k + 1, acc)

    _, acc = lax.while_loop(wcond, wbody, (k0, jnp.zeros((tb, C), jnp.float32)))
    o_ref[...] = acc


def _tc_ragged_mm(xg, w, ends, mi, e_pad):
    tb = 1024
    norm = mi is not None
    w = w.astype(jnp.bfloat16)
    in_specs = [
        pl.BlockSpec((tb, C), lambda i, e: (i, 0)),
        pl.BlockSpec((K, C, C), lambda i, e: (0, 0, 0)),
    ]
    operands = [xg, w]
    if norm:
        in_specs.append(pl.BlockSpec((8, C), lambda i, e: (0, 0)))
        operands.append(mi)
    grid_spec = pltpu.PrefetchScalarGridSpec(
        num_scalar_prefetch=1,
        grid=(e_pad // tb,),
        in_specs=in_specs,
        out_specs=pl.BlockSpec((tb, C), lambda i, e: (i, 0)),
    )
    return pl.pallas_call(
        functools.partial(_mm_body, norm=norm, tb=tb),
        grid_spec=grid_spec,
        out_shape=jax.ShapeDtypeStruct((e_pad, C), jnp.float32),
    )(ends, *operands)


def _mom_body(h_ref, o_ref, acc_ref, *, nsteps):
    i = pl.program_id(0)
    h = h_ref[...]
    blk = jnp.concatenate(
        [
            jnp.sum(h, axis=0, keepdims=True),
            jnp.sum(h * h, axis=0, keepdims=True),
            jnp.zeros((6, C), jnp.float32),
        ],
        axis=0,
    )

    @pl.when(i == 0)
    def _():
        acc_ref[...] = blk

    @pl.when(i > 0)
    def _():
        acc_ref[...] = acc_ref[...] + blk

    @pl.when(i == nsteps - 1)
    def _():
        o_ref[...] = acc_ref[...]


def _tc_moments(h, n_out):
    tb = 2048
    nsteps = n_out // tb
    return pl.pallas_call(
        functools.partial(_mom_body, nsteps=nsteps),
        grid=(nsteps,),
        in_specs=[pl.BlockSpec((tb, C), lambda i: (i, 0))],
        out_specs=pl.BlockSpec((8, C), lambda i: (0, 0)),
        out_shape=jax.ShapeDtypeStruct((8, C), jnp.float32),
        scratch_shapes=[pltpu.VMEM((8, C), jnp.float32)],
    )(h)


def _lr_body(x_ref, o_ref):
    x = x_ref[...]
    o_ref[...] = jnp.where(x >= 0, x, 0.2 * x)


def _tc_leaky(o, n):
    tb = 1000
    return pl.pallas_call(
        _lr_body,
        grid=(n // tb,),
        in_specs=[pl.BlockSpec((tb, C), lambda i: (i, 0))],
        out_specs=pl.BlockSpec((tb, C), lambda i: (i, 0)),
        out_shape=jax.ShapeDtypeStruct((n, C), jnp.float32),
    )(o)


def kernel(x_feat, W1, W2, src, dst, counts):
    n = x_feat.shape[0]
    e = src.shape[0]
    e_pad = -(-e // (NW * CHUNK * 8)) * (NW * CHUNK * 8)
    n_out = -(-(n + 1) // (NS * 512)) * (NS * 512)

    src = src.astype(jnp.int32)
    dst = dst.astype(jnp.int32)
    ends = jnp.cumsum(counts).astype(jnp.int32)
    srcp = jnp.concatenate([src, jnp.zeros((e_pad - e,), jnp.int32)])
    srcp = srcp.reshape(e_pad // CHUNK, CHUNK)
    dstp = jnp.concatenate([dst, jnp.full((e_pad - e,), n, jnp.int32)])
    dstp = dstp.reshape(e_pad // CHUNK, CHUNK)

    xg1 = _sc_gather(x_feat, srcp, e_pad)
    msg1 = _tc_ragged_mm(xg1, W1, ends, None, e_pad)
    h = _sc_scatter(msg1, dstp, e_pad, n_out)
    mom = _tc_moments(h, n_out)
    mean = mom[0] / n
    inv = lax.rsqrt(mom[1] / n - mean * mean + 1e-5)
    mi = jnp.zeros((8, C), jnp.float32).at[0].set(mean).at[1].set(inv)
    xg2 = _sc_gather(h, srcp, e_pad)
    msg2 = _tc_ragged_mm(xg2, W2, ends, mi, e_pad)
    o = _sc_scatter(msg2, dstp, e_pad, n_out)
    return _tc_leaky(o, n)


# speedup vs baseline: 1.1441x; 1.0515x over previous
import functools

import jax
import jax.numpy as jnp
from jax import lax
from jax.experimental import pallas as pl
from jax.experimental.pallas import tpu as pltpu
from jax.experimental.pallas import tpu_sc as plsc

K = 27
C = 128
NC = 2
NS = 16
NW = NC * NS
CHUNK = 128
FB = 4

_mesh = lambda: plsc.VectorSubcoreMesh(core_axis_name="c", subcore_axis_name="s")


def _sc_gather(table, idx2, e_pad):
    nch = e_pad // CHUNK
    cpw0 = nch * 6 // (7 * NS) // 8 * 8
    cpw1 = nch // NS - cpw0

    def body(tab_ref, idx_ref, out_ref, idxv, rows, sg):
        c = lax.axis_index("c")
        s = lax.axis_index("s")

        def flow(base, cpw):
            pltpu.sync_copy(idx_ref.at[pl.ds(base, cpw)], idxv.at[pl.ds(0, cpw)])

            def step(i, carry):
                cps = [
                    pltpu.async_copy(tab_ref.at[idxv.at[i * FB + b]], rows.at[b], sg)
                    for b in range(FB)
                ]
                for b in range(FB):
                    cps[b].wait()
                for b in range(FB):
                    pltpu.sync_copy(
                        rows.at[b],
                        out_ref.at[pl.ds((base + i * FB + b) * CHUNK, CHUNK)],
                    )
                return carry

            lax.fori_loop(0, cpw // FB, step, 0)

        @pl.when(c == 1)
        def _():
            flow(s * cpw0, cpw0)

        if cpw1 > 0:

            @pl.when(c == 0)
            def _():
                flow(NS * cpw0 + s * cpw1, cpw1)

    return pl.kernel(
        body,
        out_type=jax.ShapeDtypeStruct((e_pad, C), jnp.float32),
        mesh=_mesh(),
        scratch_types=[
            pltpu.VMEM((cpw0, CHUNK), jnp.int32),
            pltpu.VMEM((FB, CHUNK, C), jnp.float32),
            pltpu.SemaphoreType.DMA,
        ],
    )(table, idx2)


def _sc_scatter(msg, dst2, e_pad, n_out):
    nchunks = e_pad // CHUNK
    cpt = nchunks // NS
    rb = NS * 512
    nb = n_out // rb
    rpt = rb // NS

    def body(msg_ref, dst_ref, out_ref, zv, mv0, mv1, iv2, dstv,
             bmn, bmx, hl, spm, sg0, sg1):
        c = lax.axis_index("c")
        s = lax.axis_index("s")
        pltpu.sync_copy(dst_ref.at[pl.ds(s * cpt, cpt)], dstv)

        def z1(i, carry):
            for q in range(8):
                zv[i, pl.ds(q * 16, 16)] = jnp.zeros((16,), jnp.float32)
            return carry

        lax.fori_loop(0, CHUNK, z1, 0)

        def mm(j, carry):
            mn = jnp.int32(2147483647)
            mx = jnp.int32(-1)
            for q in range(8):
                lv = dstv[j, pl.ds(q * 16, 16)]
                mn = jnp.minimum(mn, jnp.min(lv))
                mx = jnp.maximum(mx, jnp.max(lv))
            bmn[j] = mn
            bmx[j] = mx
            return carry

        lax.fori_loop(0, cpt, mm, 0)

        for bi in range(-(-nb // NC)):
            b = bi * NC + c
            base = b * rb
            live = b < nb
            for z in range(rpt // CHUNK):
                pltpu.sync_copy(zv, spm.at[pl.ds(s * rpt + z * CHUNK, CHUNK)])
            plsc.subcore_barrier()

            def hlist(j, m):
                hit = jnp.logical_and(bmn[j] < base + rb, bmx[j] >= base)
                hl[m] = j
                return m + jnp.where(hit, jnp.int32(1), jnp.int32(0))

            m = lax.fori_loop(0, cpt, hlist, jnp.int32(0))

            def fire(i, mv, sg):
                ch = s * cpt + hl[i]
                pltpu.async_copy(msg_ref.at[pl.ds(ch * CHUNK, CHUNK)], mv, sg)

            @pl.when(m > 0)
            def _():
                fire(0, mv0, sg0)

            def proc(i, mv, sg, nmv, nsg):
                @pl.when(i + 1 < m)
                def _():
                    fire(jnp.minimum(i + 1, cpt - 1), nmv, nsg)

                ch = s * cpt + hl[i]
                pltpu.make_async_copy(
                    msg_ref.at[pl.ds(ch * CHUNK, CHUNK)], mv, sg
                ).wait()
                j = hl[i]
                for q in range(8):
                    lv = dstv[j, pl.ds(q * 16, 16)] - base
                    ok = jnp.logical_and(lv >= 0, lv < rb)
                    iv2[pl.ds(q * 16, 16)] = jnp.where(ok, lv, rb)
                pltpu.sync_copy(mv, spm.at[iv2], add=True)

            def pbody(i, carry):
                @pl.when(i % 2 == 0)
                def _():
                    proc(i, mv0, sg0, mv1, sg1)

                @pl.when(i % 2 == 1)
                def _():
                    proc(i, mv1, sg1, mv0, sg0)

                return carry

            lax.fori_loop(0, m, pbody, 0)
            plsc.subcore_barrier()

            def wb(w, carry):
                r0 = s * rpt + w * CHUNK
                pltpu.sync_copy(
                    spm.at[pl.ds(r0, CHUNK)],
                    out_ref.at[pl.ds(base + r0, CHUNK)],
                )
                return carry

            @pl.when(live)
            def _():
                lax.fori_loop(0, rpt // CHUNK, wb, 0)

            plsc.subcore_barrier()

    return pl.kernel(
        body,
        out_type=jax.ShapeDtypeStruct((n_out, C), jnp.float32),
        mesh=_mesh(),
        compiler_params=pltpu.CompilerParams(needs_layout_passes=False),
        scratch_types=[
            pltpu.VMEM((CHUNK, C), jnp.float32),
            pltpu.VMEM((CHUNK, C), jnp.float32),
            pltpu.VMEM((CHUNK, C), jnp.float32),
            pltpu.VMEM((CHUNK,), jnp.int32),
            pltpu.VMEM((cpt, CHUNK), jnp.int32),
            pltpu.SMEM((cpt,), jnp.int32),
            pltpu.SMEM((cpt,), jnp.int32),
            pltpu.SMEM((cpt + 1,), jnp.int32),
            pltpu.VMEM_SHARED((rb + 8, C), jnp.float32),
            pltpu.SemaphoreType.DMA,
            pltpu.SemaphoreType.DMA,
        ],
    )(msg, dst2)


def _mm_body(ends_ref, *refs, norm, tb):
    if norm:
        x_ref, w_ref, mi_ref, o_ref = refs
    else:
        x_ref, w_ref, o_ref = refs
    r0 = pl.program_id(0) * tb
    x = x_ref[...]
    if norm:
        t = (x - mi_ref[0:1, :]) * mi_ref[1:2, :]
        x = jnp.where(t >= 0, t, 0.2 * t)

    def cnt(j, a):
        return a + jnp.where(ends_ref[j] <= r0, 1, 0)

    k0 = lax.fori_loop(0, K, cnt, 0)
    rows = r0 + lax.broadcasted_iota(jnp.int32, (tb, 1), 0)

    def start_of(k):
        return jnp.where(k == 0, 0, ends_ref[jnp.maximum(k, 1) - 1])

    def wcond(st):
        k, _ = st
        return jnp.logical_and(k < K, start_of(jnp.minimum(k, K - 1)) < r0 + tb)

    def wbody(st):
        k, acc = st
        kc = jnp.minimum(k, K - 1)
        m = (rows >= start_of(kc)) & (rows < ends_ref[kc])
        xm = jnp.where(m, x, 0.0).astype(jnp.bfloat16)
        acc = acc + jnp.dot(xm, w_ref[kc], preferred_element_type=jnp.float32)
        return (k + 1, acc)

    _, acc = lax.while_loop(wcond, wbody, (k0, jnp.zeros((tb, C), jnp.float32)))
    o_ref[...] = acc


def _tc_ragged_mm(xg, w, ends, mi, e_pad):
    tb = 2048
    norm = mi is not None
    w = w.astype(jnp.bfloat16)
    in_specs = [
        pl.BlockSpec((tb, C), lambda i, e: (i, 0)),
        pl.BlockSpec((K, C, C), lambda i, e: (0, 0, 0)),
    ]
    operands = [xg, w]
    if norm:
        in_specs.append(pl.BlockSpec((8, C), lambda i, e: (0, 0)))
        operands.append(mi)
    grid_spec = pltpu.PrefetchScalarGridSpec(
        num_scalar_prefetch=1,
        grid=(e_pad // tb,),
        in_specs=in_specs,
        out_specs=pl.BlockSpec((tb, C), lambda i, e: (i, 0)),
    )
    return pl.pallas_call(
        functools.partial(_mm_body, norm=norm, tb=tb),
        grid_spec=grid_spec,
        out_shape=jax.ShapeDtypeStruct((e_pad, C), jnp.float32),
    )(ends, *operands)


def _mom_body(h_ref, o_ref, acc_ref, *, nsteps):
    i = pl.program_id(0)
    h = h_ref[...]
    blk = jnp.concatenate(
        [
            jnp.sum(h, axis=0, keepdims=True),
            jnp.sum(h * h, axis=0, keepdims=True),
            jnp.zeros((6, C), jnp.float32),
        ],
        axis=0,
    )

    @pl.when(i == 0)
    def _():
        acc_ref[...] = blk

    @pl.when(i > 0)
    def _():
        acc_ref[...] = acc_ref[...] + blk

    @pl.when(i == nsteps - 1)
    def _():
        o_ref[...] = acc_ref[...]


def _tc_moments(h, n_out):
    tb = 2048
    nsteps = n_out // tb
    return pl.pallas_call(
        functools.partial(_mom_body, nsteps=nsteps),
        grid=(nsteps,),
        in_specs=[pl.BlockSpec((tb, C), lambda i: (i, 0))],
        out_specs=pl.BlockSpec((8, C), lambda i: (0, 0)),
        out_shape=jax.ShapeDtypeStruct((8, C), jnp.float32),
        scratch_shapes=[pltpu.VMEM((8, C), jnp.float32)],
    )(h)


def _lr_body(x_ref, o_ref):
    x = x_ref[...]
    o_ref[...] = jnp.where(x >= 0, x, 0.2 * x)


def _tc_leaky(o, n):
    tb = 1000
    return pl.pallas_call(
        _lr_body,
        grid=(n // tb,),
        in_specs=[pl.BlockSpec((tb, C), lambda i: (i, 0))],
        out_specs=pl.BlockSpec((tb, C), lambda i: (i, 0)),
        out_shape=jax.ShapeDtypeStruct((n, C), jnp.float32),
    )(o)


def kernel(x_feat, W1, W2, src, dst, counts):
    n = x_feat.shape[0]
    e = src.shape[0]
    e_pad = -(-e // (NW * CHUNK * 8)) * (NW * CHUNK * 8)
    n_out = -(-(n + 1) // (NS * 512)) * (NS * 512)

    src = src.astype(jnp.int32)
    dst = dst.astype(jnp.int32)
    ends = jnp.cumsum(counts).astype(jnp.int32)
    srcp = jnp.concatenate([src, jnp.zeros((e_pad - e,), jnp.int32)])
    srcp = srcp.reshape(e_pad // CHUNK, CHUNK)
    dstp = jnp.concatenate([dst, jnp.full((e_pad - e,), n, jnp.int32)])
    dstp = dstp.reshape(e_pad // CHUNK, CHUNK)

    xg1 = _sc_gather(x_feat, srcp, e_pad)
    msg1 = _tc_ragged_mm(xg1, W1, ends, None, e_pad)
    h = _sc_scatter(msg1, dstp, e_pad, n_out)
    mom = _tc_moments(h, n_out)
    mean = mom[0] / n
    inv = lax.rsqrt(mom[1] / n - mean * mean + 1e-5)
    mi = jnp.zeros((8, C), jnp.float32).at[0].set(mean).at[1].set(inv)
    xg2 = _sc_gather(h, srcp, e_pad)
    msg2 = _tc_ragged_mm(xg2, W2, ends, mi, e_pad)
    o = _sc_scatter(msg2, dstp, e_pad, n_out)
    return _tc_leaky(o, n)


# speedup vs baseline: 1.1685x; 1.0213x over previous
import functools

import jax
import jax.numpy as jnp
from jax import lax
from jax.experimental import pallas as pl
from jax.experimental.pallas import tpu as pltpu
from jax.experimental.pallas import tpu_sc as plsc

K = 27
C = 128
NC = 2
NS = 16
NW = NC * NS
CHUNK = 128
FB = 4

_mesh = lambda: plsc.VectorSubcoreMesh(core_axis_name="c", subcore_axis_name="s")


def _sc_gather(table, idx2, e_pad):
    nch = e_pad // CHUNK
    cpw0 = nch * 6 // (7 * NS) // 8 * 8
    cpw1 = nch // NS - cpw0

    def body(tab_ref, idx_ref, out_ref, idxv, rows, sg):
        c = lax.axis_index("c")
        s = lax.axis_index("s")

        def flow(base, cpw):
            pltpu.sync_copy(idx_ref.at[pl.ds(base, cpw)], idxv.at[pl.ds(0, cpw)])

            def step(i, carry):
                cps = [
                    pltpu.async_copy(tab_ref.at[idxv.at[i * FB + b]], rows.at[b], sg)
                    for b in range(FB)
                ]
                for b in range(FB):
                    cps[b].wait()
                for b in range(FB):
                    pltpu.sync_copy(
                        rows.at[b],
                        out_ref.at[pl.ds((base + i * FB + b) * CHUNK, CHUNK)],
                    )
                return carry

            lax.fori_loop(0, cpw // FB, step, 0)

        @pl.when(c == 1)
        def _():
            flow(s * cpw0, cpw0)

        if cpw1 > 0:

            @pl.when(c == 0)
            def _():
                flow(NS * cpw0 + s * cpw1, cpw1)

    return pl.kernel(
        body,
        out_type=jax.ShapeDtypeStruct((e_pad, C), jnp.float32),
        mesh=_mesh(),
        scratch_types=[
            pltpu.VMEM((cpw0, CHUNK), jnp.int32),
            pltpu.VMEM((FB, CHUNK, C), jnp.float32),
            pltpu.SemaphoreType.DMA,
        ],
    )(table, idx2)


def _sc_scatter(msg, dst2, e_pad, n_out):
    nchunks = e_pad // CHUNK
    cpt = nchunks // NS
    rb = NS * 512
    nb = n_out // rb
    rpt = rb // NS

    def body(msg_ref, dst_ref, out_ref, zv, mv0, mv1, iv2, dstv,
             bmn, bmx, hl, spm, sg0, sg1):
        c = lax.axis_index("c")
        s = lax.axis_index("s")
        pltpu.sync_copy(dst_ref.at[pl.ds(s * cpt, cpt)], dstv)

        def z1(i, carry):
            for q in range(8):
                zv[i, pl.ds(q * 16, 16)] = jnp.zeros((16,), jnp.float32)
            return carry

        lax.fori_loop(0, CHUNK, z1, 0)

        def mm(j, carry):
            mn = jnp.int32(2147483647)
            mx = jnp.int32(-1)
            for q in range(8):
                lv = dstv[j, pl.ds(q * 16, 16)]
                mn = jnp.minimum(mn, jnp.min(lv))
                mx = jnp.maximum(mx, jnp.max(lv))
            bmn[j] = mn
            bmx[j] = mx
            return carry

        lax.fori_loop(0, cpt, mm, 0)

        for bi in range(-(-nb // NC)):
            b = bi * NC + c
            base = b * rb
            live = b < nb
            for z in range(rpt // CHUNK):
                pltpu.sync_copy(zv, spm.at[pl.ds(s * rpt + z * CHUNK, CHUNK)])
            plsc.subcore_barrier()

            def hlist(j, m):
                hit = jnp.logical_and(bmn[j] < base + rb, bmx[j] >= base)
                hl[m] = j
                return m + jnp.where(hit, jnp.int32(1), jnp.int32(0))

            m = lax.fori_loop(0, cpt, hlist, jnp.int32(0))

            def fire(i, mv, sg):
                ch = s * cpt + hl[i]
                pltpu.async_copy(msg_ref.at[pl.ds(ch * CHUNK, CHUNK)], mv, sg)

            @pl.when(m > 0)
            def _():
                fire(0, mv0, sg0)

            def proc(i, mv, sg, nmv, nsg):
                @pl.when(i + 1 < m)
                def _():
                    fire(jnp.minimum(i + 1, cpt - 1), nmv, nsg)

                ch = s * cpt + hl[i]
                pltpu.make_async_copy(
                    msg_ref.at[pl.ds(ch * CHUNK, CHUNK)], mv, sg
                ).wait()
                j = hl[i]
                for q in range(8):
                    lv = dstv[j, pl.ds(q * 16, 16)] - base
                    ok = jnp.logical_and(lv >= 0, lv < rb)
                    iv2[pl.ds(q * 16, 16)] = jnp.where(ok, lv, rb)
                pltpu.sync_copy(mv, spm.at[iv2], add=True)

            def pbody(i, carry):
                @pl.when(i % 2 == 0)
                def _():
                    proc(i, mv0, sg0, mv1, sg1)

                @pl.when(i % 2 == 1)
                def _():
                    proc(i, mv1, sg1, mv0, sg0)

                return carry

            lax.fori_loop(0, m, pbody, 0)
            plsc.subcore_barrier()

            def wb(w, carry):
                r0 = s * rpt + w * CHUNK
                pltpu.sync_copy(
                    spm.at[pl.ds(r0, CHUNK)],
                    out_ref.at[pl.ds(base + r0, CHUNK)],
                )
                return carry

            @pl.when(live)
            def _():
                lax.fori_loop(0, rpt // CHUNK, wb, 0)

            plsc.subcore_barrier()

    return pl.kernel(
        body,
        out_type=jax.ShapeDtypeStruct((n_out, C), jnp.float32),
        mesh=_mesh(),
        compiler_params=pltpu.CompilerParams(needs_layout_passes=False),
        scratch_types=[
            pltpu.VMEM((CHUNK, C), jnp.float32),
            pltpu.VMEM((CHUNK, C), jnp.float32),
            pltpu.VMEM((CHUNK, C), jnp.float32),
            pltpu.VMEM((CHUNK,), jnp.int32),
            pltpu.VMEM((cpt, CHUNK), jnp.int32),
            pltpu.SMEM((cpt,), jnp.int32),
            pltpu.SMEM((cpt,), jnp.int32),
            pltpu.SMEM((cpt + 1,), jnp.int32),
            pltpu.VMEM_SHARED((rb + 8, C), jnp.float32),
            pltpu.SemaphoreType.DMA,
            pltpu.SemaphoreType.DMA,
        ],
    )(msg, dst2)


def _mm_body(ends_ref, *refs, norm, tb):
    if norm:
        x_ref, w_ref, mi_ref, o_ref = refs
    else:
        x_ref, w_ref, o_ref = refs
    r0 = pl.program_id(0) * tb
    x = x_ref[...]
    if norm:
        t = (x - mi_ref[0:1, :]) * mi_ref[1:2, :]
        x = jnp.where(t >= 0, t, 0.2 * t)

    def cnt(j, a):
        return a + jnp.where(ends_ref[j] <= r0, 1, 0)

    k0 = lax.fori_loop(0, K, cnt, 0)
    rows = r0 + lax.broadcasted_iota(jnp.int32, (tb, 1), 0)

    def start_of(k):
        return jnp.where(k == 0, 0, ends_ref[jnp.maximum(k, 1) - 1])

    def wcond(st):
        k, _ = st
        return jnp.logical_and(k < K, start_of(jnp.minimum(k, K - 1)) < r0 + tb)

    def wbody(st):
        k, acc = st
        kc = jnp.minimum(k, K - 1)
        m = (rows >= start_of(kc)) & (rows < ends_ref[kc])
        xm = jnp.where(m, x, 0.0).astype(jnp.bfloat16)
        acc = acc + jnp.dot(xm, w_ref[kc], preferred_element_type=jnp.float32)
        return (k + 1, acc)

    _, acc = lax.while_loop(wcond, wbody, (k0, jnp.zeros((tb, C), jnp.float32)))
    o_ref[...] = acc


def _tc_ragged_mm(xg, w, ends, mi, e_pad):
    tb = 4096
    norm = mi is not None
    w = w.astype(jnp.bfloat16)
    in_specs = [
        pl.BlockSpec((tb, C), lambda i, e: (i, 0)),
        pl.BlockSpec((K, C, C), lambda i, e: (0, 0, 0)),
    ]
    operands = [xg, w]
    if norm:
        in_specs.append(pl.BlockSpec((8, C), lambda i, e: (0, 0)))
        operands.append(mi)
    grid_spec = pltpu.PrefetchScalarGridSpec(
        num_scalar_prefetch=1,
        grid=(e_pad // tb,),
        in_specs=in_specs,
        out_specs=pl.BlockSpec((tb, C), lambda i, e: (i, 0)),
    )
    return pl.pallas_call(
        functools.partial(_mm_body, norm=norm, tb=tb),
        grid_spec=grid_spec,
        out_shape=jax.ShapeDtypeStruct((e_pad, C), jnp.float32),
    )(ends, *operands)


def _mom_body(h_ref, o_ref, acc_ref, *, nsteps):
    i = pl.program_id(0)
    h = h_ref[...]
    blk = jnp.concatenate(
        [
            jnp.sum(h, axis=0, keepdims=True),
            jnp.sum(h * h, axis=0, keepdims=True),
            jnp.zeros((6, C), jnp.float32),
        ],
        axis=0,
    )

    @pl.when(i == 0)
    def _():
        acc_ref[...] = blk

    @pl.when(i > 0)
    def _():
        acc_ref[...] = acc_ref[...] + blk

    @pl.when(i == nsteps - 1)
    def _():
        o_ref[...] = acc_ref[...]


def _tc_moments(h, n_out):
    tb = 4096
    nsteps = n_out // tb
    return pl.pallas_call(
        functools.partial(_mom_body, nsteps=nsteps),
        grid=(nsteps,),
        in_specs=[pl.BlockSpec((tb, C), lambda i: (i, 0))],
        out_specs=pl.BlockSpec((8, C), lambda i: (0, 0)),
        out_shape=jax.ShapeDtypeStruct((8, C), jnp.float32),
        scratch_shapes=[pltpu.VMEM((8, C), jnp.float32)],
    )(h)


def _lr_body(x_ref, o_ref):
    x = x_ref[...]
    o_ref[...] = jnp.where(x >= 0, x, 0.2 * x)


def _tc_leaky(o, n):
    tb = 1000
    return pl.pallas_call(
        _lr_body,
        grid=(n // tb,),
        in_specs=[pl.BlockSpec((tb, C), lambda i: (i, 0))],
        out_specs=pl.BlockSpec((tb, C), lambda i: (i, 0)),
        out_shape=jax.ShapeDtypeStruct((n, C), jnp.float32),
    )(o)


def kernel(x_feat, W1, W2, src, dst, counts):
    n = x_feat.shape[0]
    e = src.shape[0]
    e_pad = -(-e // (NW * CHUNK * 8)) * (NW * CHUNK * 8)
    n_out = -(-(n + 1) // (NS * 512)) * (NS * 512)

    src = src.astype(jnp.int32)
    dst = dst.astype(jnp.int32)
    ends = jnp.cumsum(counts).astype(jnp.int32)
    srcp = jnp.concatenate([src, jnp.zeros((e_pad - e,), jnp.int32)])
    srcp = srcp.reshape(e_pad // CHUNK, CHUNK)
    dstp = jnp.concatenate([dst, jnp.full((e_pad - e,), n, jnp.int32)])
    dstp = dstp.reshape(e_pad // CHUNK, CHUNK)

    xg1 = _sc_gather(x_feat, srcp, e_pad)
    msg1 = _tc_ragged_mm(xg1, W1, ends, None, e_pad)
    h = _sc_scatter(msg1, dstp, e_pad, n_out)
    mom = _tc_moments(h, n_out)
    mean = mom[0] / n
    inv = lax.rsqrt(mom[1] / n - mean * mean + 1e-5)
    mi = jnp.zeros((8, C), jnp.float32).at[0].set(mean).at[1].set(inv)
    xg2 = _sc_gather(h, srcp, e_pad)
    msg2 = _tc_ragged_mm(xg2, W2, ends, mi, e_pad)
    o = _sc_scatter(msg2, dstp, e_pad, n_out)
    return _tc_leaky(o, n)
